# Initial kernel scaffold; baseline (speedup 1.0000x reference)
import jax
import jax.numpy as jnp
from jax.experimental import pallas as pl


def kernel(h_src, h_dst, edge_index, Wq, Wk, a):
    raise NotImplementedError("write your pallas kernel here")


# speedup vs baseline: 4.9258x; 4.9258x over previous
import functools

import jax
import jax.numpy as jnp
from jax import lax
from jax.experimental import pallas as pl
from jax.experimental.pallas import tpu as pltpu
from jax.experimental.pallas import tpu_sc as plsc

N_NODES = 10000
IN_DIM = 128
H = 4
DH = 32
NEG_SLOPE = 0.2
N_EDGES = 320000

NUM_WORKERS = 32
EDGES_PER_WORKER = N_EDGES // NUM_WORKERS
CHUNK = 2000
NUM_CHUNKS = EDGES_PER_WORKER // CHUNK
GROUPS = CHUNK // 16

ROW_BLOCK = 2000


def _tc_body(hs_ref, hd_ref, cq_ref, ck_ref, el_ref, er_ref):
    el_ref[...] = jnp.dot(hs_ref[...], cq_ref[...],
                          preferred_element_type=jnp.float32)
    er_ref[...] = jnp.dot(hd_ref[...], ck_ref[...],
                          preferred_element_type=jnp.float32)


def _node_scores(h_src, h_dst, cq, ck):
    grid = N_NODES // ROW_BLOCK
    return pl.pallas_call(
        _tc_body,
        grid=(grid,),
        in_specs=[
            pl.BlockSpec((ROW_BLOCK, IN_DIM), lambda i: (i, 0)),
            pl.BlockSpec((ROW_BLOCK, IN_DIM), lambda i: (i, 0)),
            pl.BlockSpec((IN_DIM, H), lambda i: (0, 0)),
            pl.BlockSpec((IN_DIM, H), lambda i: (0, 0)),
        ],
        out_specs=[
            pl.BlockSpec((ROW_BLOCK, H), lambda i: (i, 0)),
            pl.BlockSpec((ROW_BLOCK, H), lambda i: (i, 0)),
        ],
        out_shape=[
            jax.ShapeDtypeStruct((N_NODES, H), jnp.float32),
            jax.ShapeDtypeStruct((N_NODES, H), jnp.float32),
        ],
    )(h_src, h_dst, cq, ck)


_SC_MESH = plsc.VectorSubcoreMesh(core_axis_name="c", subcore_axis_name="s")


@functools.partial(
    pl.kernel,
    out_type=jax.ShapeDtypeStruct((N_EDGES * H,), jnp.float32),
    mesh=_SC_MESH,
    compiler_params=pltpu.CompilerParams(needs_layout_passes=False),
    scratch_types=[
        pltpu.VMEM((N_NODES * H,), jnp.float32),
        pltpu.VMEM((N_NODES * H,), jnp.float32),
        pltpu.VMEM((CHUNK,), jnp.int32),
        pltpu.VMEM((CHUNK,), jnp.int32),
        pltpu.VMEM((CHUNK * H,), jnp.float32),
    ],
)
def _sc_gather(el_hbm, er_hbm, src_hbm, dst_hbm, out_hbm,
               el_v, er_v, src_v, dst_v, out_v):
    cid = lax.axis_index("c")
    sid = lax.axis_index("s")
    wid = sid * 2 + cid
    pltpu.sync_copy(el_hbm, el_v)
    pltpu.sync_copy(er_hbm, er_v)
    base = wid * EDGES_PER_WORKER
    iota = lax.iota(jnp.int32, 16)

    def chunk_body(ci, carry):
        cb = pl.multiple_of(base + ci * CHUNK, 8)
        pltpu.sync_copy(src_hbm.at[pl.ds(cb, CHUNK)], src_v)
        pltpu.sync_copy(dst_hbm.at[pl.ds(cb, CHUNK)], dst_v)

        def grp(gi, carry2):
            off = pl.multiple_of(gi * 16, 16)
            s = src_v[pl.ds(off, 16)] * H
            d = dst_v[pl.ds(off, 16)] * H
            lane = (iota + off) * H
            for h in range(H):
                hv = jnp.full((16,), h, jnp.int32)
                x = plsc.load_gather(el_v, [s + hv]) \
                    + plsc.load_gather(er_v, [d + hv])
                y = jnp.maximum(x, NEG_SLOPE * x)
                plsc.store_scatter(out_v, [lane + hv], y)
            return carry2

        lax.fori_loop(0, GROUPS, grp, 0)
        pltpu.sync_copy(out_v, out_hbm.at[pl.ds(cb * H, CHUNK * H)])
        return carry

    lax.fori_loop(0, NUM_CHUNKS, chunk_body, 0)


def kernel(h_src, h_dst, edge_index, Wq, Wk, a):
    wq3 = Wq.reshape(H, DH, IN_DIM)
    wk3 = Wk.reshape(H, DH, IN_DIM)
    cq = jnp.einsum("hdi,hd->ih", wq3, a[:, :DH])
    ck = jnp.einsum("hdi,hd->ih", wk3, a[:, DH:])
    el, er = _node_scores(h_src, h_dst, cq, ck)
    src = edge_index[0].astype(jnp.int32)
    dst = edge_index[1].astype(jnp.int32)
    out = _sc_gather(el.reshape(-1), er.reshape(-1), src, dst)
    return out.reshape(N_EDGES, H)


# speedup vs baseline: 24.5278x; 4.9794x over previous
import functools

import jax
import jax.numpy as jnp
from jax import lax
from jax.experimental import pallas as pl
from jax.experimental.pallas import tpu as pltpu
from jax.experimental.pallas import tpu_sc as plsc

N_NODES = 10000
IN_DIM = 128
H = 4
DH = 32
NEG_SLOPE = 0.2
N_EDGES = 320000

BLK = 128
NBLK = N_EDGES // BLK
NUM_WORKERS = 32
BASE_BLOCKS = NBLK // NUM_WORKERS
EXTRA_TILES = NBLK - BASE_BLOCKS * NUM_WORKERS
CB = 13
NUM_CHUNKS = BASE_BLOCKS // CB

ROW_BLOCK = 2000


def _tc_body(hs_ref, hd_ref, cqt_ref, ckt_ref, el_ref, er_ref):
    el_ref[...] = lax.dot_general(cqt_ref[...], hs_ref[...],
                                  (((1,), (1,)), ((), ())),
                                  preferred_element_type=jnp.float32)
    er_ref[...] = lax.dot_general(ckt_ref[...], hd_ref[...],
                                  (((1,), (1,)), ((), ())),
                                  preferred_element_type=jnp.float32)


def _node_scores(h_src, h_dst, cqt, ckt):
    return pl.pallas_call(
        _tc_body,
        out_shape=[
            jax.ShapeDtypeStruct((H, N_NODES), jnp.float32),
            jax.ShapeDtypeStruct((H, N_NODES), jnp.float32),
        ],
    )(h_src, h_dst, cqt, ckt)


_SC_MESH = plsc.VectorSubcoreMesh(core_axis_name="c", subcore_axis_name="s")


@functools.partial(
    pl.kernel,
    out_type=jax.ShapeDtypeStruct((N_EDGES * H,), jnp.float32),
    mesh=_SC_MESH,
    compiler_params=pltpu.CompilerParams(needs_layout_passes=False),
    scratch_types=[
        pltpu.VMEM((H, N_NODES), jnp.float32),
        pltpu.VMEM((H, N_NODES), jnp.float32),
        pltpu.VMEM((CB * BLK,), jnp.int32),
        pltpu.VMEM((CB * BLK,), jnp.int32),
        pltpu.VMEM((CB * H * BLK,), jnp.float32),
    ],
)
def _sc_gather(el_hbm, er_hbm, ei_hbm, out_hbm, el_v, er_v, src_v, dst_v, out_v):
    cid = lax.axis_index("c")
    sid = lax.axis_index("s")
    wid = sid * 2 + cid
    pltpu.sync_copy(el_hbm, el_v)
    pltpu.sync_copy(er_hbm, er_v)
    start_blk = BASE_BLOCKS * wid + jnp.minimum(wid, EXTRA_TILES)

    def do_block(j, base512):
        for g in range(BLK // 16):
            s = src_v[pl.ds(pl.multiple_of(j * BLK + 16 * g, 16), 16)]
            d = dst_v[pl.ds(pl.multiple_of(j * BLK + 16 * g, 16), 16)]
            for h in range(H):
                hv = jnp.full((16,), h, jnp.int32)
                x = plsc.load_gather(el_v, [hv, s]) \
                    + plsc.load_gather(er_v, [hv, d])
                y = jnp.maximum(x, NEG_SLOPE * x)
                off = pl.multiple_of(base512 + h * BLK + 16 * g, 16)
                out_v[pl.ds(off, 16)] = y

    def chunk_body(ci, carry):
        cb_blk = start_blk + ci * CB
        e0 = pl.multiple_of(cb_blk * BLK, 8)
        pltpu.sync_copy(ei_hbm.at[0, pl.ds(e0, CB * BLK)], src_v)
        pltpu.sync_copy(ei_hbm.at[1, pl.ds(e0, CB * BLK)], dst_v)

        def blk_body(j, c2):
            do_block(j, j * H * BLK)
            return c2

        lax.fori_loop(0, CB, blk_body, 0)
        pltpu.sync_copy(
            out_v,
            out_hbm.at[pl.ds(pl.multiple_of(cb_blk * H * BLK, 8), CB * H * BLK)])
        return carry

    lax.fori_loop(0, NUM_CHUNKS, chunk_body, 0)

    @pl.when(wid < EXTRA_TILES)
    def _():
        eb = start_blk + BASE_BLOCKS
        e0 = pl.multiple_of(eb * BLK, 8)
        pltpu.sync_copy(ei_hbm.at[0, pl.ds(e0, BLK)], src_v.at[pl.ds(0, BLK)])
        pltpu.sync_copy(ei_hbm.at[1, pl.ds(e0, BLK)], dst_v.at[pl.ds(0, BLK)])
        do_block(0, 0)
        pltpu.sync_copy(
            out_v.at[pl.ds(0, H * BLK)],
            out_hbm.at[pl.ds(pl.multiple_of(eb * H * BLK, 8), H * BLK)])


def kernel(h_src, h_dst, edge_index, Wq, Wk, a):
    wq3 = Wq.reshape(H, DH, IN_DIM)
    wk3 = Wk.reshape(H, DH, IN_DIM)
    cqt = jnp.einsum("hdi,hd->hi", wq3, a[:, :DH])
    ckt = jnp.einsum("hdi,hd->hi", wk3, a[:, DH:])
    elt, ert = _node_scores(h_src, h_dst, cqt, ckt)
    out_flat = _sc_gather(elt, ert, edge_index.astype(jnp.int32))
    return (out_flat.reshape(NBLK, H, BLK).transpose(0, 2, 1)
            .reshape(N_EDGES, H))


# speedup vs baseline: 26.5749x; 1.0835x over previous
import functools

import jax
import jax.numpy as jnp
from jax import lax
from jax.experimental import pallas as pl
from jax.experimental.pallas import tpu as pltpu
from jax.experimental.pallas import tpu_sc as plsc

N_NODES = 10000
IN_DIM = 128
H = 4
DH = 32
NEG_SLOPE = 0.2
N_EDGES = 320000

BLK = 128
NBLK = N_EDGES // BLK
NUM_WORKERS = 32
BASE_BLOCKS = NBLK // NUM_WORKERS
EXTRA_TILES = NBLK - BASE_BLOCKS * NUM_WORKERS
CB = 13
NUM_CHUNKS = BASE_BLOCKS // CB

ROW_BLOCK = 2000


def _tc_body(hs_ref, hd_ref, cqt_ref, ckt_ref, el_ref, er_ref):
    el_ref[...] = lax.dot_general(cqt_ref[...], hs_ref[...],
                                  (((1,), (1,)), ((), ())),
                                  preferred_element_type=jnp.float32)
    er_ref[...] = lax.dot_general(ckt_ref[...], hd_ref[...],
                                  (((1,), (1,)), ((), ())),
                                  preferred_element_type=jnp.float32)


def _node_scores(h_src, h_dst, cqt, ckt):
    return pl.pallas_call(
        _tc_body,
        out_shape=[
            jax.ShapeDtypeStruct((H, N_NODES), jnp.float32),
            jax.ShapeDtypeStruct((H, N_NODES), jnp.float32),
        ],
    )(h_src, h_dst, cqt, ckt)


_SC_MESH = plsc.VectorSubcoreMesh(core_axis_name="c", subcore_axis_name="s")


@functools.partial(
    pl.kernel,
    out_type=jax.ShapeDtypeStruct((N_EDGES * H,), jnp.float32),
    mesh=_SC_MESH,
    compiler_params=pltpu.CompilerParams(needs_layout_passes=False),
    scratch_types=[
        pltpu.VMEM((H, N_NODES), jnp.float32),
        pltpu.VMEM((H, N_NODES), jnp.float32),
        pltpu.VMEM((2, CB * BLK), jnp.int32),
        pltpu.VMEM((2, CB * BLK), jnp.int32),
        pltpu.VMEM((2, CB * H * BLK), jnp.float32),
        pltpu.SemaphoreType.DMA,
        pltpu.SemaphoreType.DMA,
        pltpu.SemaphoreType.DMA,
        pltpu.SemaphoreType.DMA,
        pltpu.SemaphoreType.DMA,
    ],
)
def _sc_gather(el_hbm, er_hbm, ei_hbm, out_hbm, el_v, er_v, src_v, dst_v,
               out_v, sem_tab, sem_i0, sem_i1, sem_o0, sem_o1):
    cid = lax.axis_index("c")
    sid = lax.axis_index("s")
    wid = sid * 2 + cid
    start_blk = BASE_BLOCKS * wid + jnp.minimum(wid, EXTRA_TILES)

    t_el = pltpu.async_copy(el_hbm, el_v, sem_tab)
    t_er = pltpu.async_copy(er_hbm, er_v, sem_tab)

    def issue_idx(cb_blk, buf, sem):
        e0 = pl.multiple_of(cb_blk * BLK, 8)
        a = pltpu.async_copy(ei_hbm.at[0, pl.ds(e0, CB * BLK)],
                             src_v.at[buf], sem)
        b = pltpu.async_copy(ei_hbm.at[1, pl.ds(e0, CB * BLK)],
                             dst_v.at[buf], sem)
        return a, b

    i0 = issue_idx(start_blk, 0, sem_i0)
    i1 = issue_idx(start_blk + CB, 1, sem_i1)
    t_el.wait()
    t_er.wait()

    def do_block(j, buf, base512):
        for g in range(BLK // 16):
            s = src_v[buf, pl.ds(pl.multiple_of(j * BLK + 16 * g, 16), 16)]
            d = dst_v[buf, pl.ds(pl.multiple_of(j * BLK + 16 * g, 16), 16)]
            for h in range(H):
                hv = jnp.full((16,), h, jnp.int32)
                x = plsc.load_gather(el_v, [hv, s]) \
                    + plsc.load_gather(er_v, [hv, d])
                y = jnp.maximum(x, NEG_SLOPE * x)
                off = pl.multiple_of(base512 + h * BLK + 16 * g, 16)
                out_v[buf, pl.ds(off, 16)] = y

    def compute_chunk(buf):
        def blk_body(j, c2):
            do_block(j, buf, j * H * BLK)
            return c2
        lax.fori_loop(0, CB, blk_body, 0)

    def issue_out(cb_blk, buf, sem):
        return pltpu.async_copy(
            out_v.at[buf],
            out_hbm.at[pl.ds(pl.multiple_of(cb_blk * H * BLK, 8), CB * H * BLK)],
            sem)

    def drain_out(buf, sem):
        pltpu.make_async_copy(
            out_v.at[buf],
            out_hbm.at[pl.ds(0, CB * H * BLK)],
            sem).wait()

    def wait_idx(buf, sem):
        pltpu.make_async_copy(ei_hbm.at[0, pl.ds(0, CB * BLK)],
                              src_v.at[buf], sem).wait()
        pltpu.make_async_copy(ei_hbm.at[1, pl.ds(0, CB * BLK)],
                              dst_v.at[buf], sem).wait()

    def pair_body(k, carry):
        c0 = start_blk + (2 * k) * CB
        c1 = c0 + CB

        @pl.when(k > 0)
        def _():
            issue_idx(c0, 0, sem_i0)
            issue_idx(c1, 1, sem_i1)

        wait_idx(0, sem_i0)

        @pl.when(k > 0)
        def _():
            drain_out(0, sem_o0)
        compute_chunk(0)
        issue_out(c0, 0, sem_o0)

        wait_idx(1, sem_i1)

        @pl.when(k > 0)
        def _():
            drain_out(1, sem_o1)
        compute_chunk(1)
        issue_out(c1, 1, sem_o1)
        return carry

    lax.fori_loop(0, NUM_CHUNKS // 2, pair_body, 0)
    drain_out(0, sem_o0)
    drain_out(1, sem_o1)

    @pl.when(wid < EXTRA_TILES)
    def _():
        eb = start_blk + BASE_BLOCKS
        e0 = pl.multiple_of(eb * BLK, 8)
        pltpu.sync_copy(ei_hbm.at[0, pl.ds(e0, BLK)],
                        src_v.at[0, pl.ds(0, BLK)])
        pltpu.sync_copy(ei_hbm.at[1, pl.ds(e0, BLK)],
                        dst_v.at[0, pl.ds(0, BLK)])
        do_block(0, 0, 0)
        pltpu.sync_copy(
            out_v.at[0, pl.ds(0, H * BLK)],
            out_hbm.at[pl.ds(pl.multiple_of(eb * H * BLK, 8), H * BLK)])


def kernel(h_src, h_dst, edge_index, Wq, Wk, a):
    wq3 = Wq.reshape(H, DH, IN_DIM)
    wk3 = Wk.reshape(H, DH, IN_DIM)
    cqt = jnp.einsum("hdi,hd->hi", wq3, a[:, :DH])
    ckt = jnp.einsum("hdi,hd->hi", wk3, a[:, DH:])
    elt, ert = _node_scores(h_src, h_dst, cqt, ckt)
    out_flat = _sc_gather(elt, ert, edge_index.astype(jnp.int32))
    return (out_flat.reshape(NBLK, H, BLK).transpose(0, 2, 1)
            .reshape(N_EDGES, H))


# speedup vs baseline: 41.8514x; 1.5748x over previous
import functools

import jax
import jax.numpy as jnp
from jax import lax
from jax.experimental import pallas as pl
from jax.experimental.pallas import tpu as pltpu
from jax.experimental.pallas import tpu_sc as plsc

N_NODES = 10000
IN_DIM = 128
H = 4
DH = 32
NEG_SLOPE = 0.2
N_EDGES = 320000

BLK = 128
NBLK = N_EDGES // BLK
NUM_WORKERS = 32
BASE_BLOCKS = NBLK // NUM_WORKERS
EXTRA_TILES = NBLK - BASE_BLOCKS * NUM_WORKERS
CB = 13
NUM_CHUNKS = BASE_BLOCKS // CB

ROW_BLOCK = 2000


def _tc_body(hs_ref, hd_ref, cqt_ref, ckt_ref, el_ref, er_ref):
    el_ref[...] = lax.dot_general(cqt_ref[...], hs_ref[...],
                                  (((1,), (1,)), ((), ())),
                                  preferred_element_type=jnp.float32)
    er_ref[...] = lax.dot_general(ckt_ref[...], hd_ref[...],
                                  (((1,), (1,)), ((), ())),
                                  preferred_element_type=jnp.float32)


def _node_scores(h_src, h_dst, cqt, ckt):
    return pl.pallas_call(
        _tc_body,
        out_shape=[
            jax.ShapeDtypeStruct((H, N_NODES), jnp.float32),
            jax.ShapeDtypeStruct((H, N_NODES), jnp.float32),
        ],
    )(h_src, h_dst, cqt, ckt)


_SC_MESH = plsc.VectorSubcoreMesh(core_axis_name="c", subcore_axis_name="s")


@functools.partial(
    pl.kernel,
    out_type=jax.ShapeDtypeStruct((N_EDGES * H,), jnp.float32),
    mesh=_SC_MESH,
    compiler_params=pltpu.CompilerParams(needs_layout_passes=False),
    scratch_types=[
        pltpu.VMEM((H, N_NODES), jnp.float32),
        pltpu.VMEM((H, N_NODES), jnp.float32),
        pltpu.VMEM((2, CB * BLK), jnp.int32),
        pltpu.VMEM((2, CB * BLK), jnp.int32),
        pltpu.VMEM((2, CB * H * BLK), jnp.float32),
        pltpu.SemaphoreType.DMA,
        pltpu.SemaphoreType.DMA,
        pltpu.SemaphoreType.DMA,
        pltpu.SemaphoreType.DMA,
        pltpu.SemaphoreType.DMA,
    ],
)
def _sc_gather(el_hbm, er_hbm, ei_hbm, out_hbm, el_v, er_v, src_v, dst_v,
               out_v, sem_tab, sem_i0, sem_i1, sem_o0, sem_o1):
    cid = lax.axis_index("c")
    sid = lax.axis_index("s")
    wid = sid * 2 + cid
    start_blk = BASE_BLOCKS * wid + jnp.minimum(wid, EXTRA_TILES)

    t_el = pltpu.async_copy(el_hbm, el_v, sem_tab)
    t_er = pltpu.async_copy(er_hbm, er_v, sem_tab)

    def issue_idx(cb_blk, buf, sem):
        e0 = pl.multiple_of(cb_blk * BLK, 8)
        a = pltpu.async_copy(ei_hbm.at[0, pl.ds(e0, CB * BLK)],
                             src_v.at[buf], sem)
        b = pltpu.async_copy(ei_hbm.at[1, pl.ds(e0, CB * BLK)],
                             dst_v.at[buf], sem)
        return a, b

    i0 = issue_idx(start_blk, 0, sem_i0)
    i1 = issue_idx(start_blk + CB, 1, sem_i1)
    t_el.wait()
    t_er.wait()

    def do_block(j, buf, base512):
        for gp in range(BLK // 32):
            vals = []
            for g in (2 * gp, 2 * gp + 1):
                s = src_v[buf, pl.ds(pl.multiple_of(j * BLK + 16 * g, 16), 16)]
                d = dst_v[buf, pl.ds(pl.multiple_of(j * BLK + 16 * g, 16), 16)]
                for h in range(H):
                    hv = jnp.full((16,), h, jnp.int32)
                    vals.append((g, h,
                                 plsc.load_gather(el_v, [hv, s]),
                                 plsc.load_gather(er_v, [hv, d])))
            for g, h, a, b in vals:
                x = a + b
                y = jnp.maximum(x, NEG_SLOPE * x)
                off = pl.multiple_of(base512 + h * BLK + 16 * g, 16)
                out_v[buf, pl.ds(off, 16)] = y

    def compute_chunk(buf):
        @functools.partial(plsc.parallel_loop, 0, CB)
        def blk_body(j):
            do_block(j, buf, j * H * BLK)

    def issue_out(cb_blk, buf, sem):
        return pltpu.async_copy(
            out_v.at[buf],
            out_hbm.at[pl.ds(pl.multiple_of(cb_blk * H * BLK, 8), CB * H * BLK)],
            sem)

    def drain_out(buf, sem):
        pltpu.make_async_copy(
            out_v.at[buf],
            out_hbm.at[pl.ds(0, CB * H * BLK)],
            sem).wait()

    def wait_idx(buf, sem):
        pltpu.make_async_copy(ei_hbm.at[0, pl.ds(0, CB * BLK)],
                              src_v.at[buf], sem).wait()
        pltpu.make_async_copy(ei_hbm.at[1, pl.ds(0, CB * BLK)],
                              dst_v.at[buf], sem).wait()

    def pair_body(k, carry):
        c0 = start_blk + (2 * k) * CB
        c1 = c0 + CB

        @pl.when(k > 0)
        def _():
            issue_idx(c0, 0, sem_i0)
            issue_idx(c1, 1, sem_i1)

        wait_idx(0, sem_i0)

        @pl.when(k > 0)
        def _():
            drain_out(0, sem_o0)
        compute_chunk(0)
        issue_out(c0, 0, sem_o0)

        wait_idx(1, sem_i1)

        @pl.when(k > 0)
        def _():
            drain_out(1, sem_o1)
        compute_chunk(1)
        issue_out(c1, 1, sem_o1)
        return carry

    lax.fori_loop(0, NUM_CHUNKS // 2, pair_body, 0)
    drain_out(0, sem_o0)
    drain_out(1, sem_o1)

    @pl.when(wid < EXTRA_TILES)
    def _():
        eb = start_blk + BASE_BLOCKS
        e0 = pl.multiple_of(eb * BLK, 8)
        pltpu.sync_copy(ei_hbm.at[0, pl.ds(e0, BLK)],
                        src_v.at[0, pl.ds(0, BLK)])
        pltpu.sync_copy(ei_hbm.at[1, pl.ds(e0, BLK)],
                        dst_v.at[0, pl.ds(0, BLK)])
        do_block(0, 0, 0)
        pltpu.sync_copy(
            out_v.at[0, pl.ds(0, H * BLK)],
            out_hbm.at[pl.ds(pl.multiple_of(eb * H * BLK, 8), H * BLK)])


def kernel(h_src, h_dst, edge_index, Wq, Wk, a):
    wq3 = Wq.reshape(H, DH, IN_DIM)
    wk3 = Wk.reshape(H, DH, IN_DIM)
    cqt = jnp.einsum("hdi,hd->hi", wq3, a[:, :DH])
    ckt = jnp.einsum("hdi,hd->hi", wk3, a[:, DH:])
    elt, ert = _node_scores(h_src, h_dst, cqt, ckt)
    out_flat = _sc_gather(elt, ert, edge_index.astype(jnp.int32))
    return (out_flat.reshape(NBLK, H, BLK).transpose(0, 2, 1)
            .reshape(N_EDGES, H))
